# Initial kernel scaffold; baseline (speedup 1.0000x reference)
import jax
import jax.numpy as jnp
from jax.experimental import pallas as pl


def kernel(indices, table):
    raise NotImplementedError("write your pallas kernel here")


# speedup vs baseline: 1.2575x; 1.2575x over previous
import functools

import jax
import jax.numpy as jnp
from jax import lax
from jax.experimental import pallas as pl
from jax.experimental.pallas import tpu as pltpu
from jax.experimental.pallas import tpu_sc as plsc

D = 128
NW = 32
CHUNK = 128
GROUP = 4
GROUP_ROWS = GROUP * CHUNK


def _make_gather(n_rows: int, vocab: int):
    b_per_w = n_rows // NW
    n_chunks = b_per_w // CHUNK
    n_groups = n_chunks // GROUP
    mesh = plsc.VectorSubcoreMesh(core_axis_name="c", subcore_axis_name="s")

    @functools.partial(
        pl.kernel,
        mesh=mesh,
        out_type=jax.ShapeDtypeStruct((n_rows, D), jnp.float32),
        scratch_types=[
            pltpu.VMEM((n_chunks, CHUNK), jnp.int32),
            pltpu.VMEM((GROUP_ROWS, D), jnp.float32),
            pltpu.SemaphoreType.DMA,
        ],
    )
    def gather_kernel(idx_hbm, table_hbm, out_hbm, idx_v, rows_v, sem):
        wid = lax.axis_index("s") * 2 + lax.axis_index("c")
        pltpu.sync_copy(idx_hbm.at[wid], idx_v)

        def body(g, carry):
            copies = []
            for j in range(GROUP):
                copies.append(pltpu.async_copy(
                    table_hbm.at[idx_v.at[g * GROUP + j]],
                    rows_v.at[pl.ds(j * CHUNK, CHUNK)],
                    sem))
            for cp in copies:
                cp.wait()
            row_base = wid * b_per_w + g * GROUP_ROWS
            pltpu.sync_copy(rows_v, out_hbm.at[pl.ds(row_base, GROUP_ROWS)])
            return carry

        lax.fori_loop(0, n_groups, body, 0)

    return gather_kernel


def kernel(indices, table):
    batch, fields = indices.shape
    n_rows = batch * fields
    idx = indices.astype(jnp.int32).reshape(NW, n_rows // (NW * CHUNK), CHUNK)
    out = _make_gather(n_rows, table.shape[0])(idx, table)
    return out.reshape(batch, fields, D)


# speedup vs baseline: 1.2852x; 1.0220x over previous
import functools

import jax
import jax.numpy as jnp
from jax import lax
from jax.experimental import pallas as pl
from jax.experimental.pallas import tpu as pltpu
from jax.experimental.pallas import tpu_sc as plsc

D = 128
NW = 32
CHUNK = 128
GROUP = 2
GROUP_ROWS = GROUP * CHUNK


def _make_gather(n_rows: int):
    b_per_w = n_rows // NW
    n_chunks = b_per_w // CHUNK
    n_groups = n_chunks // GROUP
    half = n_groups // 2
    mesh = plsc.VectorSubcoreMesh(core_axis_name="c", subcore_axis_name="s")

    @functools.partial(
        pl.kernel,
        mesh=mesh,
        out_type=jax.ShapeDtypeStruct((n_rows, D), jnp.float32),
        scratch_types=[
            pltpu.VMEM((n_chunks, CHUNK), jnp.int32),
            pltpu.VMEM((GROUP_ROWS, D), jnp.float32),
            pltpu.VMEM((GROUP_ROWS, D), jnp.float32),
            pltpu.SemaphoreType.DMA,
            pltpu.SemaphoreType.DMA,
        ],
    )
    def gather_kernel(idx_hbm, table_hbm, out_hbm, idx_v, rows_a, rows_b,
                      sem_a, sem_b):
        wid = lax.axis_index("s") * 2 + lax.axis_index("c")
        pltpu.sync_copy(idx_hbm.at[wid], idx_v)
        base = wid * b_per_w

        def gathers(g, buf, sem):
            return [pltpu.make_async_copy(
                        table_hbm.at[idx_v.at[g * GROUP + j]],
                        buf.at[pl.ds(j * CHUNK, CHUNK)],
                        sem)
                    for j in range(GROUP)]

        def fire(g, buf, sem):
            for cp in gathers(g, buf, sem):
                cp.start()

        def drain(g, buf, sem):
            for cp in gathers(g, buf, sem):
                cp.wait()

        def write_out(g, buf):
            pltpu.sync_copy(buf, out_hbm.at[pl.ds(base + g * GROUP_ROWS,
                                                  GROUP_ROWS)])

        fire(0, rows_a, sem_a)

        def body(h, carry):
            g = 2 * h
            fire(g + 1, rows_b, sem_b)
            drain(g, rows_a, sem_a)
            write_out(g, rows_a)
            fire(g + 2, rows_a, sem_a)
            drain(g + 1, rows_b, sem_b)
            write_out(g + 1, rows_b)
            return carry

        lax.fori_loop(0, half - 1, body, 0)

        g = n_groups - 2
        fire(g + 1, rows_b, sem_b)
        drain(g, rows_a, sem_a)
        write_out(g, rows_a)
        drain(g + 1, rows_b, sem_b)
        write_out(g + 1, rows_b)

    return gather_kernel


def kernel(indices, table):
    batch, fields = indices.shape
    n_rows = batch * fields
    idx = indices.astype(jnp.int32).reshape(NW, n_rows // (NW * CHUNK), CHUNK)
    out = _make_gather(n_rows)(idx, table)
    return out.reshape(batch, fields, D)


# speedup vs baseline: 2.2356x; 1.7396x over previous
import functools

import jax
import jax.numpy as jnp
from jax import lax
from jax.experimental import pallas as pl
from jax.experimental.pallas import tpu as pltpu
from jax.experimental.pallas import tpu_sc as plsc

D = 128
NW = 32
GROUP = 4


def _make_gather(batch: int, fields: int):
    epw = batch // NW
    n_groups = epw // GROUP
    half = n_groups // 2
    mesh = plsc.VectorSubcoreMesh(core_axis_name="c", subcore_axis_name="s")

    @functools.partial(
        pl.kernel,
        mesh=mesh,
        out_type=jax.ShapeDtypeStruct((batch, fields, D), jnp.float32),
        scratch_types=[
            pltpu.VMEM((epw, fields), jnp.int32),
            pltpu.VMEM((GROUP, fields, D), jnp.float32),
            pltpu.VMEM((GROUP, fields, D), jnp.float32),
            pltpu.SemaphoreType.DMA,
            pltpu.SemaphoreType.DMA,
        ],
        compiler_params=pltpu.CompilerParams(use_tc_tiling_on_sc=True),
    )
    def gather_kernel(idx_hbm, table_hbm, out_hbm, idx_v, rows_a, rows_b,
                      sem_a, sem_b):
        wid = lax.axis_index("s") * 2 + lax.axis_index("c")
        ebase = wid * epw
        pltpu.sync_copy(idx_hbm.at[pl.ds(ebase, epw)], idx_v)

        def gathers(g, buf, sem):
            return [pltpu.make_async_copy(
                        table_hbm.at[idx_v.at[g * GROUP + j]],
                        buf.at[j],
                        sem)
                    for j in range(GROUP)]

        def fire(g, buf, sem):
            for cp in gathers(g, buf, sem):
                cp.start()

        def drain(g, buf, sem):
            for cp in gathers(g, buf, sem):
                cp.wait()

        def write_out(g, buf):
            pltpu.sync_copy(buf, out_hbm.at[pl.ds(ebase + g * GROUP, GROUP)])

        fire(0, rows_a, sem_a)

        def body(h, carry):
            g = 2 * h
            fire(g + 1, rows_b, sem_b)
            drain(g, rows_a, sem_a)
            write_out(g, rows_a)
            fire(g + 2, rows_a, sem_a)
            drain(g + 1, rows_b, sem_b)
            write_out(g + 1, rows_b)
            return carry

        lax.fori_loop(0, half - 1, body, 0)

        g = n_groups - 2
        fire(g + 1, rows_b, sem_b)
        drain(g, rows_a, sem_a)
        write_out(g, rows_a)
        drain(g + 1, rows_b, sem_b)
        write_out(g + 1, rows_b)

    return gather_kernel


def kernel(indices, table):
    batch, fields = indices.shape
    return _make_gather(batch, fields)(indices.astype(jnp.int32), table)


# speedup vs baseline: 4.0827x; 1.8262x over previous
import functools

import jax
import jax.numpy as jnp
from jax import lax
from jax.experimental import pallas as pl
from jax.experimental.pallas import tpu as pltpu
from jax.experimental.pallas import tpu_sc as plsc

D = 128
NW = 32
BBLK = 128
GROUP = 2


def _make_gather(batch: int, fields: int):
    n_groups = fields // GROUP
    half = n_groups // 2
    mesh = plsc.VectorSubcoreMesh(core_axis_name="c", subcore_axis_name="s")

    @functools.partial(
        pl.kernel,
        mesh=mesh,
        out_type=jax.ShapeDtypeStruct((fields, batch, D), jnp.float32),
        scratch_types=[
            pltpu.VMEM((fields, BBLK), jnp.int32),
            pltpu.VMEM((GROUP, BBLK, D), jnp.float32),
            pltpu.VMEM((GROUP, BBLK, D), jnp.float32),
            pltpu.SemaphoreType.DMA,
            pltpu.SemaphoreType.DMA,
        ],
        compiler_params=pltpu.CompilerParams(use_tc_tiling_on_sc=True),
    )
    def gather_kernel(idx_hbm, table_hbm, out_hbm, idx_v, rows_a, rows_b,
                      sem_a, sem_b):
        wid = lax.axis_index("s") * 2 + lax.axis_index("c")
        b0 = wid * BBLK
        pltpu.sync_copy(idx_hbm.at[:, pl.ds(b0, BBLK)], idx_v)

        def gathers(g, buf, sem):
            return [pltpu.make_async_copy(
                        table_hbm.at[idx_v.at[g * GROUP + j]],
                        buf.at[j],
                        sem)
                    for j in range(GROUP)]

        def fire(g, buf, sem):
            for cp in gathers(g, buf, sem):
                cp.start()

        def drain(g, buf, sem):
            for cp in gathers(g, buf, sem):
                cp.wait()

        def write_out(g, buf):
            pltpu.sync_copy(buf, out_hbm.at[pl.ds(g * GROUP, GROUP),
                                            pl.ds(b0, BBLK)])

        fire(0, rows_a, sem_a)

        def body(h, carry):
            g = 2 * h
            fire(g + 1, rows_b, sem_b)
            drain(g, rows_a, sem_a)
            write_out(g, rows_a)
            fire(g + 2, rows_a, sem_a)
            drain(g + 1, rows_b, sem_b)
            write_out(g + 1, rows_b)
            return carry

        lax.fori_loop(0, half - 1, body, 0)

        g = n_groups - 2
        fire(g + 1, rows_b, sem_b)
        drain(g, rows_a, sem_a)
        write_out(g, rows_a)
        drain(g + 1, rows_b, sem_b)
        write_out(g + 1, rows_b)

    return gather_kernel


def kernel(indices, table):
    batch, fields = indices.shape
    idx_t = indices.T.astype(jnp.int32)
    out_t = _make_gather(batch, fields)(idx_t, table)
    return out_t.transpose(1, 0, 2)
